# Initial kernel scaffold; baseline (speedup 1.0000x reference)
import jax
import jax.numpy as jnp
from jax.experimental import pallas as pl


def kernel(input_ids, embeddings):
    raise NotImplementedError("write your pallas kernel here")


# speedup vs baseline: 3.3108x; 3.3108x over previous
import functools

import jax
import jax.numpy as jnp
from jax import lax
from jax.experimental import pallas as pl
from jax.experimental.pallas import tpu as pltpu
from jax.experimental.pallas import tpu_sc as plsc


@functools.cache
def _make_gather(B, D, V):
    info = plsc.get_sparse_core_info()
    NC, NS = info.num_cores, info.num_subcores
    NW = NC * NS
    assert B % NW == 0
    b_per_w = B // NW
    C = 800
    assert b_per_w % C == 0
    n_chunks = b_per_w // C
    mesh = plsc.VectorSubcoreMesh(core_axis_name="c", subcore_axis_name="s")

    @functools.partial(
        pl.kernel,
        mesh=mesh,
        out_type=jax.ShapeDtypeStruct((B, D), jnp.float32),
        scratch_types=[
            pltpu.VMEM((C,), jnp.int32),
            pltpu.VMEM((C, D), jnp.float32),
            pltpu.SemaphoreType.DMA,
        ],
    )
    def gather_kernel(table_hbm, idx_hbm, out_hbm, idx_v, rows_v, sem):
        wid = lax.axis_index("s") * NC + lax.axis_index("c")
        base = wid * b_per_w

        def body(j, carry):
            off = base + j * C
            pltpu.sync_copy(idx_hbm.at[pl.ds(off, C)], idx_v)
            pltpu.async_copy(table_hbm.at[idx_v], rows_v, sem).wait()
            pltpu.sync_copy(rows_v, out_hbm.at[pl.ds(off, C)])
            return carry

        lax.fori_loop(0, n_chunks, body, 0)

    return gather_kernel


def kernel(input_ids, embeddings):
    batch, hist = input_ids.shape
    vocab, dim = embeddings.shape
    idx = input_ids.reshape(-1).astype(jnp.int32)
    out = _make_gather(batch * hist, dim, vocab)(embeddings, idx)
    return out.reshape(batch, hist, dim)


# speedup vs baseline: 3.3399x; 1.0088x over previous
import functools

import jax
import jax.numpy as jnp
from jax import lax
from jax.experimental import pallas as pl
from jax.experimental.pallas import tpu as pltpu
from jax.experimental.pallas import tpu_sc as plsc


@functools.cache
def _make_gather(B, D, V):
    info = plsc.get_sparse_core_info()
    NC, NS = info.num_cores, info.num_subcores
    NW = NC * NS
    assert B % NW == 0
    b_per_w = B // NW
    C = 400
    assert b_per_w % C == 0
    n_chunks = b_per_w // C
    mesh = plsc.VectorSubcoreMesh(core_axis_name="c", subcore_axis_name="s")

    @functools.partial(
        pl.kernel,
        mesh=mesh,
        out_type=jax.ShapeDtypeStruct((B, D), jnp.float32),
        scratch_types=[
            pltpu.VMEM((C,), jnp.int32),
            pltpu.VMEM((C,), jnp.int32),
            pltpu.VMEM((C, D), jnp.float32),
            pltpu.VMEM((C, D), jnp.float32),
            pltpu.SemaphoreType.DMA,
            pltpu.SemaphoreType.DMA,
        ],
    )
    def gather_kernel(table_hbm, idx_hbm, out_hbm, idx0, idx1, rows0, rows1,
                      gsem, osem):
        idx_v = (idx0, idx1)
        rows_v = (rows0, rows1)
        wid = lax.axis_index("s") * NC + lax.axis_index("c")
        base = wid * b_per_w

        gathers = [None] * n_chunks
        outs = [None] * n_chunks
        for j in range(n_chunks):
            b = j & 1
            if j >= 2:
                outs[j - 2].wait()
            off = base + j * C
            pltpu.sync_copy(idx_hbm.at[pl.ds(off, C)], idx_v[b])
            gathers[j] = pltpu.async_copy(table_hbm.at[idx_v[b]], rows_v[b], gsem)
            if j >= 1:
                gathers[j - 1].wait()
                outs[j - 1] = pltpu.async_copy(
                    rows_v[(j - 1) & 1], out_hbm.at[pl.ds(base + (j - 1) * C, C)],
                    osem)
        gathers[n_chunks - 1].wait()
        outs[n_chunks - 1] = pltpu.async_copy(
            rows_v[(n_chunks - 1) & 1],
            out_hbm.at[pl.ds(base + (n_chunks - 1) * C, C)], osem)
        outs[n_chunks - 2].wait()
        outs[n_chunks - 1].wait()

    return gather_kernel


def kernel(input_ids, embeddings):
    batch, hist = input_ids.shape
    vocab, dim = embeddings.shape
    idx = input_ids.reshape(-1).astype(jnp.int32)
    out = _make_gather(batch * hist, dim, vocab)(embeddings, idx)
    return out.reshape(batch, hist, dim)
